# baseline (speedup 1.0000x reference)
import functools

import jax
import jax.numpy as jnp
from jax import lax
from jax.experimental import pallas as pl
from jax.experimental.pallas import tpu as pltpu
from jax.experimental.pallas import tpu_sc as plsc

D = 128


def _proj_body(t_ref, w_ref, b_ref, o_ref):
    o_ref[...] = lax.dot_general(
        jnp.maximum(t_ref[...], 0.0), w_ref[...],
        dimension_numbers=(((1,), (1,)), ((), ())),
        preferred_element_type=jnp.float32,
    ) + b_ref[...]


def _proj_table(table, W, b):
    V = table.shape[0]
    RB = 512
    return pl.pallas_call(
        _proj_body,
        grid=(pl.cdiv(V, RB),),
        in_specs=[
            pl.BlockSpec((RB, D), lambda i: (i, 0)),
            pl.BlockSpec((D, D), lambda i: (0, 0)),
            pl.BlockSpec((1, D), lambda i: (0, 0)),
        ],
        out_specs=pl.BlockSpec((RB, D), lambda i: (i, 0)),
        out_shape=jax.ShapeDtypeStruct((V, D), jnp.float32),
    )(table, W, b.reshape(1, D))


_INFO = plsc.get_sparse_core_info()
_NC, _NS = _INFO.num_cores, _INFO.num_subcores
_NW = _NC * _NS


_NBUF = 4


@functools.lru_cache(maxsize=None)
def _make_gather(V, counts):
    max_rows = max(counts)
    mesh = plsc.VectorSubcoreMesh(core_axis_name="c", subcore_axis_name="s")

    out_type = tuple(
        jax.ShapeDtypeStruct((c * _NW * 128, D), jnp.float32) for c in counts
    ) * 4

    @functools.partial(
        pl.kernel,
        out_type=out_type,
        mesh=mesh,
        scratch_types=[pltpu.VMEM((max_rows * 128,), jnp.int32)]
        + [pltpu.VMEM((128, D), jnp.float32) for _ in range(_NBUF)]
        + [pltpu.SemaphoreType.DMA for _ in range(_NBUF)],
    )
    def gather(table_hbm, proj_hbm, i0, i1, i2, i3,
               ex0, ex1, ex2, ex3, px0, px1, px2, px3,
               ey0, ey1, ey2, ey3, py0, py1, py2, py3,
               idx_v, *bufs_sems):
        bufs = bufs_sems[:_NBUF]
        sems = bufs_sems[_NBUF:]
        wid = lax.axis_index("s") * _NC + lax.axis_index("c")
        idx_refs = (i0, i1, i2, i3)
        emb_outs = ((ex0, ey0), (ex1, ey1), (ex2, ey2), (ex3, ey3))
        proj_outs = ((px0, py0), (px1, py1), (px2, py2), (px3, py3))

        def pipeline(tbl, outs, nr, base_r):
            def issue(j, b):
                pltpu.async_copy(
                    tbl.at[idx_v.at[pl.ds(j * 128, 128)]], bufs[b], sems[b])

            def retire(j, b):
                pltpu.make_async_copy(
                    tbl.at[pl.ds(0, 128)], bufs[b], sems[b]).wait()
                for out in outs:
                    pltpu.sync_copy(
                        bufs[b], out.at[pl.ds((base_r + j) * 128, 128)])

            for b in range(_NBUF):
                issue(b, b)
            steady = nr - _NBUF
            nk = (steady + _NBUF - 1) // _NBUF

            def body(k, _):
                for b in range(_NBUF):
                    j = k * _NBUF + b

                    @pl.when(j < steady)
                    def _(j=j, b=b):
                        retire(j, b)
                        issue(j + _NBUF, b)
                return 0

            lax.fori_loop(0, nk, body, 0)
            for j in range(max(steady, 0), nr):
                retire(j, j % _NBUF)

        for seg in range(4):
            nr = counts[seg]
            base_r = wid * nr
            pltpu.sync_copy(idx_refs[seg].at[pl.ds(base_r * 128, nr * 128)],
                            idx_v.at[pl.ds(0, nr * 128)])
            pipeline(table_hbm, emb_outs[seg], nr, base_r)
            pipeline(proj_hbm, proj_outs[seg], nr, base_r)

    return gather


def kernel(tensor_demo, tensor_med, tensor_vitals, tensor_labs, table, W, b):
    V = table.shape[0]
    proj_tab = _proj_table(table, W, b)

    idxs = []
    shapes = []
    counts = []
    for t in (tensor_demo, tensor_med, tensor_vitals, tensor_labs):
        shapes.append(t.shape)
        n = t.shape[0] * t.shape[1]
        counts.append(n // (128 * _NW))
        idxs.append(t.astype(jnp.int32).reshape(n))

    outs = _make_gather(V, tuple(counts))(table, proj_tab, *idxs)
    groups = [
        tuple(o.reshape(s[0], s[1], D)
              for o, s in zip(outs[4 * g:4 * g + 4], shapes))
        for g in range(4)
    ]
    return tuple(groups)


# speedup vs baseline: 1.0042x; 1.0042x over previous
import functools

import jax
import jax.numpy as jnp
from jax import lax
from jax.experimental import pallas as pl
from jax.experimental.pallas import tpu as pltpu
from jax.experimental.pallas import tpu_sc as plsc

D = 128


def _proj_body(t_ref, w_ref, b_ref, o_ref):
    o_ref[...] = lax.dot_general(
        jnp.maximum(t_ref[...], 0.0), w_ref[...],
        dimension_numbers=(((1,), (1,)), ((), ())),
        preferred_element_type=jnp.float32,
    ) + b_ref[...]


def _proj_table(table, W, b):
    V = table.shape[0]
    RB = 512
    return pl.pallas_call(
        _proj_body,
        grid=(pl.cdiv(V, RB),),
        in_specs=[
            pl.BlockSpec((RB, D), lambda i: (i, 0)),
            pl.BlockSpec((D, D), lambda i: (0, 0)),
            pl.BlockSpec((1, D), lambda i: (0, 0)),
        ],
        out_specs=pl.BlockSpec((RB, D), lambda i: (i, 0)),
        out_shape=jax.ShapeDtypeStruct((V, D), jnp.float32),
    )(table, W, b.reshape(1, D))


_INFO = plsc.get_sparse_core_info()
_NC, _NS = _INFO.num_cores, _INFO.num_subcores
_NW = _NC * _NS


_NBUF = 4


@functools.lru_cache(maxsize=None)
def _make_gather(V, counts):
    max_rows = max(counts)
    mesh = plsc.VectorSubcoreMesh(core_axis_name="c", subcore_axis_name="s")

    out_type = tuple(
        jax.ShapeDtypeStruct((c * _NW * 128, D), jnp.float32) for c in counts
    ) * 4

    @functools.partial(
        pl.kernel,
        out_type=out_type,
        mesh=mesh,
        scratch_types=[pltpu.VMEM((max_rows * 128,), jnp.int32)]
        + [pltpu.VMEM((128, D), jnp.float32) for _ in range(_NBUF)]
        + [pltpu.SemaphoreType.DMA for _ in range(2 * _NBUF)],
    )
    def gather(table_hbm, proj_hbm, i0, i1, i2, i3,
               ex0, ex1, ex2, ex3, px0, px1, px2, px3,
               ey0, ey1, ey2, ey3, py0, py1, py2, py3,
               idx_v, *bufs_sems):
        bufs = bufs_sems[:_NBUF]
        gsems = bufs_sems[_NBUF:2 * _NBUF]
        ssems = bufs_sems[2 * _NBUF:]
        wid = lax.axis_index("s") * _NC + lax.axis_index("c")
        idx_refs = (i0, i1, i2, i3)
        emb_outs = ((ex0, ey0), (ex1, ey1), (ex2, ey2), (ex3, ey3))
        proj_outs = ((px0, py0), (px1, py1), (px2, py2), (px3, py3))

        def pipeline(tbl, outs, nr, base_r):
            G = _NBUF // 2

            def issue(j, b):
                pltpu.async_copy(
                    tbl.at[idx_v.at[pl.ds(j * 128, 128)]], bufs[b],
                    gsems[b])

            def wait_g(b):
                pltpu.make_async_copy(
                    tbl.at[pl.ds(0, 128)], bufs[b], gsems[b]).wait()

            def store(j, b):
                for out in outs:
                    pltpu.async_copy(
                        bufs[b], out.at[pl.ds((base_r + j) * 128, 128)],
                        ssems[b])

            def wait_s(b):
                for out in outs:
                    pltpu.make_async_copy(
                        bufs[b], out.at[pl.ds(0, 128)], ssems[b]).wait()

            for b in range(G):
                issue(b, b)
            for j in range(G):
                issue(j + G, (j + G) % _NBUF)
                wait_g(j % _NBUF)
                store(j, j % _NBUF)

            steady_n = nr - 2 * G
            nk = (steady_n + _NBUF - 1) // _NBUF

            def body(k, _):
                for u in range(_NBUF):
                    j = G + k * _NBUF + u

                    @pl.when(j < nr - G)
                    def _(j=j, u=u):
                        b = (G + u) % _NBUF
                        br = (2 * G + u) % _NBUF
                        wait_s(br)
                        issue(j + G, br)
                        wait_g(b)
                        store(j, b)
                return 0

            lax.fori_loop(0, nk, body, 0)
            for j in range(nr - G, nr):
                wait_g(j % _NBUF)
                store(j, j % _NBUF)
            for j in range(nr - _NBUF, nr):
                wait_s(j % _NBUF)

        for seg in range(4):
            nr = counts[seg]
            base_r = wid * nr
            pltpu.sync_copy(idx_refs[seg].at[pl.ds(base_r * 128, nr * 128)],
                            idx_v.at[pl.ds(0, nr * 128)])
            pipeline(table_hbm, emb_outs[seg], nr, base_r)
            pipeline(proj_hbm, proj_outs[seg], nr, base_r)

    return gather


def kernel(tensor_demo, tensor_med, tensor_vitals, tensor_labs, table, W, b):
    V = table.shape[0]
    proj_tab = _proj_table(table, W, b)

    idxs = []
    shapes = []
    counts = []
    for t in (tensor_demo, tensor_med, tensor_vitals, tensor_labs):
        shapes.append(t.shape)
        n = t.shape[0] * t.shape[1]
        counts.append(n // (128 * _NW))
        idxs.append(t.astype(jnp.int32).reshape(n))

    outs = _make_gather(V, tuple(counts))(table, proj_tab, *idxs)
    groups = [
        tuple(o.reshape(s[0], s[1], D)
              for o, s in zip(outs[4 * g:4 * g + 4], shapes))
        for g in range(4)
    ]
    return tuple(groups)


# speedup vs baseline: 1.5898x; 1.5832x over previous
import functools

import jax
import jax.numpy as jnp
from jax import lax
from jax.experimental import pallas as pl
from jax.experimental.pallas import tpu as pltpu
from jax.experimental.pallas import tpu_sc as plsc

D = 128


_INFO = plsc.get_sparse_core_info()
_NC, _NS = _INFO.num_cores, _INFO.num_subcores
_NW = _NC * _NS
_NBUF = 4


@functools.lru_cache(maxsize=None)
def _make_gather(V, counts):
    max_rows = max(counts)
    mesh = plsc.VectorSubcoreMesh(core_axis_name="c", subcore_axis_name="s")

    out_type = tuple(
        jax.ShapeDtypeStruct((c * _NW * 128, D), jnp.float32) for c in counts
    )

    @functools.partial(
        pl.kernel,
        out_type=out_type,
        mesh=mesh,
        scratch_types=[pltpu.VMEM((max_rows * 128,), jnp.int32)]
        + [pltpu.VMEM((128, D), jnp.float32) for _ in range(_NBUF)]
        + [pltpu.SemaphoreType.DMA for _ in range(2 * _NBUF)],
    )
    def gather(table_hbm, i0, i1, i2, i3, o0, o1, o2, o3, idx_v, *bufs_sems):
        bufs = bufs_sems[:_NBUF]
        gsems = bufs_sems[_NBUF:2 * _NBUF]
        ssems = bufs_sems[2 * _NBUF:]
        wid = lax.axis_index("s") * _NC + lax.axis_index("c")
        idx_refs = (i0, i1, i2, i3)
        outs = (o0, o1, o2, o3)

        def pipeline(out, nr, base_r):
            G = _NBUF // 2

            def issue(j, b):
                pltpu.async_copy(
                    table_hbm.at[idx_v.at[pl.ds(j * 128, 128)]], bufs[b],
                    gsems[b])

            def wait_g(b):
                pltpu.make_async_copy(
                    table_hbm.at[pl.ds(0, 128)], bufs[b], gsems[b]).wait()

            def store(j, b):
                pltpu.async_copy(
                    bufs[b], out.at[pl.ds((base_r + j) * 128, 128)],
                    ssems[b])

            def wait_s(b):
                pltpu.make_async_copy(
                    bufs[b], out.at[pl.ds(0, 128)], ssems[b]).wait()

            for b in range(G):
                issue(b, b)
            for j in range(G):
                issue(j + G, (j + G) % _NBUF)
                wait_g(j % _NBUF)
                store(j, j % _NBUF)

            steady_n = nr - 2 * G
            nk = (steady_n + _NBUF - 1) // _NBUF

            def body(k, _):
                for u in range(_NBUF):
                    j = G + k * _NBUF + u

                    @pl.when(j < nr - G)
                    def _(j=j, u=u):
                        b = (G + u) % _NBUF
                        br = (2 * G + u) % _NBUF
                        wait_s(br)
                        issue(j + G, br)
                        wait_g(b)
                        store(j, b)
                return 0

            lax.fori_loop(0, nk, body, 0)
            for j in range(nr - G, nr):
                wait_g(j % _NBUF)
                store(j, j % _NBUF)
            for j in range(nr - _NBUF, nr):
                wait_s(j % _NBUF)

        for seg in range(4):
            nr = counts[seg]
            base_r = wid * nr
            pltpu.sync_copy(idx_refs[seg].at[pl.ds(base_r * 128, nr * 128)],
                            idx_v.at[pl.ds(0, nr * 128)])
            pipeline(outs[seg], nr, base_r)

    return gather


def _finish_body(bn, ns, rows_ref, w_ref, b_ref,
                 ex_ref, px_ref, ey_ref, py_ref):
    e = rows_ref[...]
    p = lax.dot_general(
        jnp.maximum(e, 0.0), w_ref[...],
        dimension_numbers=(((1,), (1,)), ((), ())),
        preferred_element_type=jnp.float32,
    ) + b_ref[...]
    e3 = e.reshape(bn, ns, D)
    p3 = p.reshape(bn, ns, D)
    ex_ref[...] = e3
    ey_ref[...] = e3
    px_ref[...] = p3
    py_ref[...] = p3


def _finish(rows, W, b, B, ns):
    bn = 64
    shp = jax.ShapeDtypeStruct((B, ns, D), jnp.float32)
    o3 = pl.BlockSpec((bn, ns, D), lambda i: (i, 0, 0))
    return pl.pallas_call(
        functools.partial(_finish_body, bn, ns),
        grid=(B // bn,),
        in_specs=[
            pl.BlockSpec((bn * ns, D), lambda i: (i, 0)),
            pl.BlockSpec((D, D), lambda i: (0, 0)),
            pl.BlockSpec((1, D), lambda i: (0, 0)),
        ],
        out_specs=[o3, o3, o3, o3],
        out_shape=[shp, shp, shp, shp],
    )(rows, W, b.reshape(1, D))


def kernel(tensor_demo, tensor_med, tensor_vitals, tensor_labs, table, W, b):
    V = table.shape[0]
    tensors = (tensor_demo, tensor_med, tensor_vitals, tensor_labs)
    idxs = []
    counts = []
    for t in tensors:
        n = t.shape[0] * t.shape[1]
        counts.append(n // (128 * _NW))
        idxs.append(t.astype(jnp.int32).reshape(n))

    rows = _make_gather(V, tuple(counts))(table, *idxs)

    embs_x, projs_x, embs_y, projs_y = [], [], [], []
    for r, t in zip(rows, tensors):
        ex, px, ey, py = _finish(r, W, b, t.shape[0], t.shape[1])
        embs_x.append(ex)
        projs_x.append(px)
        embs_y.append(ey)
        projs_y.append(py)
    return (tuple(embs_x), tuple(projs_x), tuple(embs_y), tuple(projs_y))
